# baseline (speedup 1.0000x reference)
import functools

import numpy as np

import jax
import jax.numpy as jnp
from jax import lax
from jax.experimental import pallas as pl
from jax.experimental.pallas import tpu as pltpu
from jax.experimental.pallas import tpu_sc as plsc

N = 10000
E = 320000
D = 128
NC = 2
NS = 16
NW = NC * NS

CH = 40
EPT = E // NW
NCH = EPT // CH
ZCH = 40
ZNCH = N // ZCH
ZPT = -(-ZNCH // NS)

ECH = 80
ENCH = N // ECH

_SC_MESH = plsc.VectorSubcoreMesh(core_axis_name="c", subcore_axis_name="s")


def _emb_body(emb_hbm, x_hbm, out_hbm, idx_v, rows_v, sem):
    c = lax.axis_index("c")
    s = lax.axis_index("s")
    wid = c * NS + s
    for i in range(4):
        cid = wid + i * NW

        @pl.when(cid < ENCH)
        def _():
            base = cid * ECH
            pltpu.sync_copy(x_hbm.at[pl.ds(base, ECH)], idx_v)
            pltpu.async_copy(emb_hbm.at[idx_v], rows_v, sem).wait()
            pltpu.sync_copy(rows_v, out_hbm.at[pl.ds(base, ECH)])


@jax.jit
def _emb_gather(emb, x):
    return pl.kernel(
        _emb_body,
        out_type=jax.ShapeDtypeStruct((N, D), jnp.float32),
        mesh=_SC_MESH,
        scratch_types=[
            pltpu.VMEM((ECH,), jnp.int32),
            pltpu.VMEM((ECH, D), jnp.float32),
            pltpu.SemaphoreType.DMA,
        ],
    )(emb, x)


def _edge_body(k_hbm, qv_hbm, src_hbm, dst_hbm, part_hbm,
               sh_agg, idx_s0, idx_d0, idx_s1, idx_d1,
               kd0, qv0, kd1, qv1, m, sem0, sem1, isem0, isem1):
    c = lax.axis_index("c")
    s = lax.axis_index("s")
    wid = c * NS + s
    ebase = wid * EPT

    idx_s = (idx_s0, idx_s1)
    idx_d = (idx_d0, idx_d1)
    kd = (kd0, kd1)
    qv = (qv0, qv1)
    sems = (sem0, sem1)
    isems = (isem0, isem1)

    zero16 = jnp.zeros((16,), jnp.float32)

    def zfill(r, carry):
        for j in range(D // 16):
            m[r, pl.ds(j * 16, 16)] = zero16
        return carry

    lax.fori_loop(0, ZCH, zfill, 0)
    for z in range(ZPT):
        cid = s + z * NS

        @pl.when(cid < ZNCH)
        def _():
            pltpu.sync_copy(m, sh_agg.at[pl.ds(cid * ZCH, ZCH)])

    plsc.subcore_barrier()

    def fetch_idx(ci, b):
        base = ebase + ci * CH
        pltpu.async_copy(src_hbm.at[pl.ds(base, CH)], idx_s[b], isems[b])
        pltpu.async_copy(dst_hbm.at[pl.ds(base, CH)], idx_d[b], isems[b])

    def wait_idx(b):
        pltpu.make_async_copy(src_hbm.at[pl.ds(0, CH)], idx_s[b], isems[b]).wait()
        pltpu.make_async_copy(dst_hbm.at[pl.ds(0, CH)], idx_d[b], isems[b]).wait()

    def fetch_rows(b):
        pltpu.async_copy(k_hbm.at[idx_d[b]], kd[b], sems[b])
        pltpu.async_copy(qv_hbm.at[idx_s[b]], qv[b], sems[b])

    def consume(b):
        pltpu.make_async_copy(k_hbm.at[idx_d[b]], kd[b], sems[b]).wait()
        pltpu.make_async_copy(qv_hbm.at[idx_s[b]], qv[b], sems[b]).wait()

        @plsc.parallel_loop(0, CH)
        def edge_body(e):
            msk = jnp.int32(-65536)
            for g in range(D // 32):
                wq = qv[b][e, pl.ds(g * 16, 16)]
                wv = qv[b][e, pl.ds(D // 2 + g * 16, 16)]
                qa = jax.lax.bitcast_convert_type(wq << 16, jnp.float32)
                qb = jax.lax.bitcast_convert_type(wq & msk, jnp.float32)
                va = jax.lax.bitcast_convert_type(wv << 16, jnp.float32)
                vb = jax.lax.bitcast_convert_type(wv & msk, jnp.float32)
                ka = kd[b][e, pl.ds(g * 32, 16)]
                kb = kd[b][e, pl.ds(g * 32 + 16, 16)]
                xa = ka + qa
                m[e, pl.ds(g * 32, 16)] = va / (1.0 + jnp.exp(-xa))
                xb = kb + qb
                m[e, pl.ds(g * 32 + 16, 16)] = vb / (1.0 + jnp.exp(-xb))

        pltpu.sync_copy(m, sh_agg.at[idx_d[b]], add=True)

    fetch_idx(0, 0)
    wait_idx(0)
    fetch_rows(0)
    fetch_idx(1, 1)

    def outer(ci2, carry):
        ci0 = ci2 * 2
        wait_idx(1)
        fetch_rows(1)
        consume(0)

        @pl.when(ci0 + 2 < NCH)
        def _():
            fetch_idx(ci0 + 2, 0)

        @pl.when(ci0 + 2 < NCH)
        def _():
            wait_idx(0)
            fetch_rows(0)

        consume(1)

        @pl.when(ci0 + 3 < NCH)
        def _():
            fetch_idx(ci0 + 3, 1)

        return carry

    lax.fori_loop(0, NCH // 2, outer, 0)
    plsc.subcore_barrier()

    for z in range(ZPT):
        cid = s + z * NS

        @pl.when(cid < ZNCH)
        def _():
            row0 = cid * ZCH
            pltpu.sync_copy(sh_agg.at[pl.ds(row0, ZCH)], m)
            pltpu.sync_copy(m, part_hbm.at[c, pl.ds(row0, ZCH)])


@jax.jit
def _edge_stage(k, qv, src, dst):
    return pl.kernel(
        _edge_body,
        out_type=jax.ShapeDtypeStruct((NC, N, D), jnp.float32),
        mesh=_SC_MESH,
        scratch_types=[
            pltpu.VMEM_SHARED((N, D), jnp.float32),
            pltpu.VMEM((CH,), jnp.int32),
            pltpu.VMEM((CH,), jnp.int32),
            pltpu.VMEM((CH,), jnp.int32),
            pltpu.VMEM((CH,), jnp.int32),
            pltpu.VMEM((CH, D), jnp.float32),
            pltpu.VMEM((CH, D), jnp.int32),
            pltpu.VMEM((CH, D), jnp.float32),
            pltpu.VMEM((CH, D), jnp.int32),
            pltpu.VMEM((CH, D), jnp.float32),
            pltpu.SemaphoreType.DMA,
            pltpu.SemaphoreType.DMA,
            pltpu.SemaphoreType.DMA,
            pltpu.SemaphoreType.DMA,
        ],
    )(k, qv, src, dst)


BLK = 400
NBLK = N // BLK


def _mm4_kernel(h_ref, wk, wq, wv, ws, bk, bq, bv, k_o, q_o, v_o, s_o):
    h = h_ref[...]
    k_o[...] = jnp.dot(h, wk[...], preferred_element_type=jnp.float32) + bk[...]
    q_o[...] = jnp.dot(h, wq[...], preferred_element_type=jnp.float32) + bq[...]
    v_o[...] = jnp.dot(h, wv[...], preferred_element_type=jnp.float32) + bv[...]
    s_o[...] = jnp.dot(h, ws[...], preferred_element_type=jnp.float32)


def _mm4_bn_kernel(t_ref, mu, var, ga, be, wk, wq, wv, ws, bk, bq, bv,
                   k_o, q_o, v_o, s_o):
    t = t_ref[...]
    h = (t - mu[...]) * jax.lax.rsqrt(var[...] + 1e-5) * ga[...] + be[...]
    h = jnp.maximum(h, 0.0)
    k_o[...] = jnp.dot(h, wk[...], preferred_element_type=jnp.float32) + bk[...]
    q_o[...] = jnp.dot(h, wq[...], preferred_element_type=jnp.float32) + bq[...]
    v_o[...] = jnp.dot(h, wv[...], preferred_element_type=jnp.float32) + bv[...]
    s_o[...] = jnp.dot(h, ws[...], preferred_element_type=jnp.float32)


_row_spec = pl.BlockSpec((BLK, D), lambda i: (i, 0))
_w_spec = pl.BlockSpec((D, D), lambda i: (0, 0))
_b_spec = pl.BlockSpec((1, D), lambda i: (0, 0))
_out4 = [jax.ShapeDtypeStruct((N, D), jnp.float32)] * 4


@jax.jit
def _mm4(h, wk, wq, wv, ws, bk, bq, bv):
    return pl.pallas_call(
        _mm4_kernel,
        grid=(NBLK,),
        in_specs=[_row_spec, _w_spec, _w_spec, _w_spec, _w_spec,
                  _b_spec, _b_spec, _b_spec],
        out_specs=[_row_spec] * 4,
        out_shape=_out4,
    )(h, wk, wq, wv, ws, bk, bq, bv)


@jax.jit
def _mm4_bn(t, mu, var, ga, be, wk, wq, wv, ws, bk, bq, bv):
    return pl.pallas_call(
        _mm4_bn_kernel,
        grid=(NBLK,),
        in_specs=[_row_spec, _b_spec, _b_spec, _b_spec, _b_spec,
                  _w_spec, _w_spec, _w_spec, _w_spec,
                  _b_spec, _b_spec, _b_spec],
        out_specs=[_row_spec] * 4,
        out_shape=_out4,
    )(t, mu, var, ga, be, wk, wq, wv, ws, bk, bq, bv)


def _tstats_kernel(p_ref, pm, s_ref, bc, t_o, mu_o, var_o, acc_sum, acc_sq):
    i = pl.program_id(0)
    agg = p_ref[0] + p_ref[1]
    t = (jnp.dot(agg, pm[...], preferred_element_type=jnp.float32)
         + s_ref[...] + bc[...])
    t_o[...] = t
    psum = jnp.sum(t, axis=0, keepdims=True)
    psq = jnp.sum(t * t, axis=0, keepdims=True)

    @pl.when(i == 0)
    def _():
        acc_sum[...] = psum
        acc_sq[...] = psq

    @pl.when(i > 0)
    def _():
        acc_sum[...] += psum
        acc_sq[...] += psq

    @pl.when(i == NBLK - 1)
    def _():
        mu = acc_sum[...] / N
        mu_o[...] = mu
        var_o[...] = acc_sq[...] / N - mu * mu


@jax.jit
def _tstats(part, pm, s, bc):
    return pl.pallas_call(
        _tstats_kernel,
        grid=(NBLK,),
        in_specs=[
            pl.BlockSpec((NC, BLK, D), lambda i: (0, i, 0)),
            _w_spec, _row_spec, _b_spec,
        ],
        out_specs=[_row_spec,
                   pl.BlockSpec((1, D), lambda i: (0, 0)),
                   pl.BlockSpec((1, D), lambda i: (0, 0))],
        out_shape=[jax.ShapeDtypeStruct((N, D), jnp.float32),
                   jax.ShapeDtypeStruct((1, D), jnp.float32),
                   jax.ShapeDtypeStruct((1, D), jnp.float32)],
        scratch_shapes=[pltpu.VMEM((1, D), jnp.float32),
                        pltpu.VMEM((1, D), jnp.float32)],
    )(part, pm, s, bc)


def _final_kernel(t_ref, mu, var, ga, be, fw, fb, out_o):
    t = t_ref[...]
    h = (t - mu[...]) * jax.lax.rsqrt(var[...] + 1e-5) * ga[...] + be[...]
    h = jnp.maximum(h, 0.0)
    out_o[...] = jnp.dot(h, fw[...], preferred_element_type=jnp.float32) + fb[...]


@jax.jit
def _final(t, mu, var, ga, be, fw, fb):
    return pl.pallas_call(
        _final_kernel,
        grid=(NBLK,),
        in_specs=[_row_spec, _b_spec, _b_spec, _b_spec, _b_spec,
                  _w_spec, _b_spec],
        out_specs=_row_spec,
        out_shape=jax.ShapeDtypeStruct((N, D), jnp.float32),
    )(t, mu, var, ga, be, fw, fb)


_TRUE_NP = np.zeros((D,), np.int64)
_PERM_NP = np.zeros((D, D), np.float32)
for _g in range(D // 32):
    for _r in range(32):
        _c = 32 * _g + (2 * _r if _r < 16 else 2 * (_r - 16) + 1)
        _TRUE_NP[32 * _g + _r] = _c
        _PERM_NP[32 * _g + _r, _c] = 1.0


def _packqv(q, v):
    qb = jax.lax.bitcast_convert_type(
        q.astype(jnp.bfloat16).reshape(N, D // 2, 2), jnp.int32)
    vb = jax.lax.bitcast_convert_type(
        v.astype(jnp.bfloat16).reshape(N, D // 2, 2), jnp.int32)
    return jnp.concatenate([qb, vb], axis=1)


def kernel(x, edge_index, emb, Wk, bk, Wq, bq, Wv, bv, Ws, bconv, gamma, beta,
           fcW, fcb):
    x = x.astype(jnp.int32)
    src = edge_index[0].astype(jnp.int32)
    dst = edge_index[1].astype(jnp.int32)
    r = lambda b: b.reshape(1, D)
    pm = jnp.asarray(_PERM_NP)
    tr = jnp.asarray(_TRUE_NP)

    h0 = _emb_gather(emb, x)
    k1, q1, v1, s1 = _mm4(h0, Wk[0][:, tr], Wq[0], Wv[0], Ws[0],
                          r(bk[0][tr]), r(bq[0]), r(bv[0]))
    p1 = _edge_stage(k1, _packqv(q1, v1), src, dst)
    t1, mu1, var1 = _tstats(p1, pm, s1, r(bconv[0]))
    k2, q2, v2, s2 = _mm4_bn(t1, mu1, var1, r(gamma[0]), r(beta[0]),
                             Wk[1][:, tr], Wq[1], Wv[1], Ws[1],
                             r(bk[1][tr]), r(bq[1]), r(bv[1]))
    p2 = _edge_stage(k2, _packqv(q2, v2), src, dst)
    t2, mu2, var2 = _tstats(p2, pm, s2, r(bconv[1]))
    return _final(t2, mu2, var2, r(gamma[1]), r(beta[1]), fcW, r(fcb))


# speedup vs baseline: 1.1550x; 1.1550x over previous
import functools

import jax
import jax.numpy as jnp
from jax import lax
from jax.experimental import pallas as pl
from jax.experimental.pallas import tpu as pltpu
from jax.experimental.pallas import tpu_sc as plsc

N = 10000
E = 320000
D = 128
NC = 2
NS = 16
NW = NC * NS

CH = 40
EPT = E // NW
NCH = EPT // CH
ZCH = 40
ZNCH = N // ZCH
ZPT = -(-ZNCH // NS)

ECH = 80
ENCH = N // ECH

_SC_MESH = plsc.VectorSubcoreMesh(core_axis_name="c", subcore_axis_name="s")


def _emb_body(emb_hbm, x_hbm, out_hbm, idx_v, rows_v, sem):
    c = lax.axis_index("c")
    s = lax.axis_index("s")
    wid = c * NS + s
    for i in range(4):
        cid = wid + i * NW

        @pl.when(cid < ENCH)
        def _():
            base = cid * ECH
            pltpu.sync_copy(x_hbm.at[pl.ds(base, ECH)], idx_v)
            pltpu.async_copy(emb_hbm.at[idx_v], rows_v, sem).wait()
            pltpu.sync_copy(rows_v, out_hbm.at[pl.ds(base, ECH)])


@jax.jit
def _emb_gather(emb, x):
    return pl.kernel(
        _emb_body,
        out_type=jax.ShapeDtypeStruct((N, D), jnp.float32),
        mesh=_SC_MESH,
        scratch_types=[
            pltpu.VMEM((ECH,), jnp.int32),
            pltpu.VMEM((ECH, D), jnp.float32),
            pltpu.SemaphoreType.DMA,
        ],
    )(emb, x)


def _edge_body(k_hbm, q_hbm, v_hbm, src_hbm, dst_hbm, part_hbm,
               sh_agg, idx_s0, idx_d0, idx_s1, idx_d1,
               kd0, qs0, vs0, kd1, qs1, vs1, m, sem0, sem1, isem0, isem1):
    c = lax.axis_index("c")
    s = lax.axis_index("s")
    wid = c * NS + s
    ebase = wid * EPT

    idx_s = (idx_s0, idx_s1)
    idx_d = (idx_d0, idx_d1)
    kd = (kd0, kd1)
    qs = (qs0, qs1)
    vs = (vs0, vs1)
    sems = (sem0, sem1)
    isems = (isem0, isem1)

    zero16 = jnp.zeros((16,), jnp.float32)

    def zfill(r, carry):
        for j in range(D // 16):
            m[r, pl.ds(j * 16, 16)] = zero16
        return carry

    lax.fori_loop(0, ZCH, zfill, 0)
    for z in range(ZPT):
        cid = s + z * NS

        @pl.when(cid < ZNCH)
        def _():
            pltpu.sync_copy(m, sh_agg.at[pl.ds(cid * ZCH, ZCH)])

    plsc.subcore_barrier()

    def fetch_idx(ci, b):
        base = ebase + ci * CH
        pltpu.async_copy(src_hbm.at[pl.ds(base, CH)], idx_s[b], isems[b])
        pltpu.async_copy(dst_hbm.at[pl.ds(base, CH)], idx_d[b], isems[b])

    def wait_idx(b):
        pltpu.make_async_copy(src_hbm.at[pl.ds(0, CH)], idx_s[b], isems[b]).wait()
        pltpu.make_async_copy(dst_hbm.at[pl.ds(0, CH)], idx_d[b], isems[b]).wait()

    def fetch_rows(b):
        pltpu.async_copy(k_hbm.at[idx_d[b]], kd[b], sems[b])
        pltpu.async_copy(q_hbm.at[idx_s[b]], qs[b], sems[b])
        pltpu.async_copy(v_hbm.at[idx_s[b]], vs[b], sems[b])

    def consume(b):
        pltpu.make_async_copy(k_hbm.at[idx_d[b]], kd[b], sems[b]).wait()
        pltpu.make_async_copy(q_hbm.at[idx_s[b]], qs[b], sems[b]).wait()
        pltpu.make_async_copy(v_hbm.at[idx_s[b]], vs[b], sems[b]).wait()

        @plsc.parallel_loop(0, CH)
        def edge_body(e):
            for j in range(D // 16):
                sl = pl.ds(j * 16, 16)
                x = kd[b][e, sl] + qs[b][e, sl]
                eta = 1.0 / (1.0 + jnp.exp(-x))
                m[e, sl] = eta * vs[b][e, sl]

        pltpu.sync_copy(m, sh_agg.at[idx_d[b]], add=True)

    fetch_idx(0, 0)
    wait_idx(0)
    fetch_rows(0)
    fetch_idx(1, 1)

    def outer(ci2, carry):
        ci0 = ci2 * 2
        wait_idx(1)
        fetch_rows(1)
        consume(0)

        @pl.when(ci0 + 2 < NCH)
        def _():
            fetch_idx(ci0 + 2, 0)

        @pl.when(ci0 + 2 < NCH)
        def _():
            wait_idx(0)
            fetch_rows(0)

        consume(1)

        @pl.when(ci0 + 3 < NCH)
        def _():
            fetch_idx(ci0 + 3, 1)

        return carry

    lax.fori_loop(0, NCH // 2, outer, 0)
    plsc.subcore_barrier()

    for z in range(ZPT):
        cid = s + z * NS

        @pl.when(cid < ZNCH)
        def _():
            row0 = cid * ZCH
            pltpu.sync_copy(sh_agg.at[pl.ds(row0, ZCH)], m)
            pltpu.sync_copy(m, part_hbm.at[c, pl.ds(row0, ZCH)])


@jax.jit
def _edge_stage(k, q, v, src, dst):
    return pl.kernel(
        _edge_body,
        out_type=jax.ShapeDtypeStruct((NC, N, D), jnp.float32),
        mesh=_SC_MESH,
        scratch_types=[
            pltpu.VMEM_SHARED((N, D), jnp.float32),
            pltpu.VMEM((CH,), jnp.int32),
            pltpu.VMEM((CH,), jnp.int32),
            pltpu.VMEM((CH,), jnp.int32),
            pltpu.VMEM((CH,), jnp.int32),
            pltpu.VMEM((CH, D), jnp.float32),
            pltpu.VMEM((CH, D), jnp.float32),
            pltpu.VMEM((CH, D), jnp.float32),
            pltpu.VMEM((CH, D), jnp.float32),
            pltpu.VMEM((CH, D), jnp.float32),
            pltpu.VMEM((CH, D), jnp.float32),
            pltpu.VMEM((CH, D), jnp.float32),
            pltpu.SemaphoreType.DMA,
            pltpu.SemaphoreType.DMA,
            pltpu.SemaphoreType.DMA,
            pltpu.SemaphoreType.DMA,
        ],
    )(k, q, v, src, dst)


BLK = 400
NBLK = N // BLK


def _mm4_kernel(h_ref, wk, wq, wv, ws, bk, bq, bv, k_o, q_o, v_o, s_o):
    h = h_ref[...]
    k_o[...] = jnp.dot(h, wk[...], preferred_element_type=jnp.float32) + bk[...]
    q_o[...] = jnp.dot(h, wq[...], preferred_element_type=jnp.float32) + bq[...]
    v_o[...] = jnp.dot(h, wv[...], preferred_element_type=jnp.float32) + bv[...]
    s_o[...] = jnp.dot(h, ws[...], preferred_element_type=jnp.float32)


def _mm4_bn_kernel(t_ref, mu, var, ga, be, wk, wq, wv, ws, bk, bq, bv,
                   k_o, q_o, v_o, s_o):
    t = t_ref[...]
    h = (t - mu[...]) * jax.lax.rsqrt(var[...] + 1e-5) * ga[...] + be[...]
    h = jnp.maximum(h, 0.0)
    k_o[...] = jnp.dot(h, wk[...], preferred_element_type=jnp.float32) + bk[...]
    q_o[...] = jnp.dot(h, wq[...], preferred_element_type=jnp.float32) + bq[...]
    v_o[...] = jnp.dot(h, wv[...], preferred_element_type=jnp.float32) + bv[...]
    s_o[...] = jnp.dot(h, ws[...], preferred_element_type=jnp.float32)


_row_spec = pl.BlockSpec((BLK, D), lambda i: (i, 0))
_w_spec = pl.BlockSpec((D, D), lambda i: (0, 0))
_b_spec = pl.BlockSpec((1, D), lambda i: (0, 0))
_out4 = [jax.ShapeDtypeStruct((N, D), jnp.float32)] * 4


@jax.jit
def _mm4(h, wk, wq, wv, ws, bk, bq, bv):
    return pl.pallas_call(
        _mm4_kernel,
        grid=(NBLK,),
        in_specs=[_row_spec, _w_spec, _w_spec, _w_spec, _w_spec,
                  _b_spec, _b_spec, _b_spec],
        out_specs=[_row_spec] * 4,
        out_shape=_out4,
    )(h, wk, wq, wv, ws, bk, bq, bv)


@jax.jit
def _mm4_bn(t, mu, var, ga, be, wk, wq, wv, ws, bk, bq, bv):
    return pl.pallas_call(
        _mm4_bn_kernel,
        grid=(NBLK,),
        in_specs=[_row_spec, _b_spec, _b_spec, _b_spec, _b_spec,
                  _w_spec, _w_spec, _w_spec, _w_spec,
                  _b_spec, _b_spec, _b_spec],
        out_specs=[_row_spec] * 4,
        out_shape=_out4,
    )(t, mu, var, ga, be, wk, wq, wv, ws, bk, bq, bv)


def _combine_stats(j, i, p_ref, s_ref, bc, t_all, acc):
    t = p_ref[0] + p_ref[1] + s_ref[...] + bc[...]
    t_all[pl.ds(i * BLK, BLK), :] = t
    psum = jnp.sum(t, axis=0, keepdims=True)
    psq = jnp.sum(t * t, axis=0, keepdims=True)

    @pl.when(i == 0)
    def _():
        acc[0:1] = psum
        acc[1:2] = psq

    @pl.when(i > 0)
    def _():
        acc[0:1] += psum
        acc[1:2] += psq


def _bn_from(t_all, acc, i, ga, be):
    mu = acc[0:1] / N
    var = acc[1:2] / N - mu * mu
    t = t_all[pl.ds(i * BLK, BLK), :]
    h = (t - mu) * jax.lax.rsqrt(var + 1e-5) * ga[...] + be[...]
    return jnp.maximum(h, 0.0)


def _comb_mm4_kernel(p_ref, s_ref, bc, ga, be, wk, wq, wv, ws, bk, bq, bv,
                     k_o, q_o, v_o, s_o, t_all, acc):
    j = pl.program_id(0)
    i = pl.program_id(1)

    @pl.when(j == 0)
    def _():
        _combine_stats(j, i, p_ref, s_ref, bc, t_all, acc)

    @pl.when(j == 1)
    def _():
        h = _bn_from(t_all, acc, i, ga, be)
        k_o[...] = jnp.dot(h, wk[...], preferred_element_type=jnp.float32) + bk[...]
        q_o[...] = jnp.dot(h, wq[...], preferred_element_type=jnp.float32) + bq[...]
        v_o[...] = jnp.dot(h, wv[...], preferred_element_type=jnp.float32) + bv[...]
        s_o[...] = jnp.dot(h, ws[...], preferred_element_type=jnp.float32)


_pj_spec = pl.BlockSpec((NC, BLK, D), lambda j, i: (0, i * (1 - j), 0))
_rowj_spec = pl.BlockSpec((BLK, D), lambda j, i: (i * (1 - j), 0))
_rowo_spec = pl.BlockSpec((BLK, D), lambda j, i: (i, 0))
_wj_spec = pl.BlockSpec((D, D), lambda j, i: (0, 0))
_bj_spec = pl.BlockSpec((1, D), lambda j, i: (0, 0))


@jax.jit
def _comb_mm4(part, s, bc, ga, be, wk, wq, wv, ws, bk, bq, bv):
    return pl.pallas_call(
        _comb_mm4_kernel,
        grid=(2, NBLK),
        in_specs=[_pj_spec, _rowj_spec, _bj_spec, _bj_spec, _bj_spec,
                  _wj_spec, _wj_spec, _wj_spec, _wj_spec,
                  _bj_spec, _bj_spec, _bj_spec],
        out_specs=[_rowo_spec] * 4,
        out_shape=_out4,
        scratch_shapes=[pltpu.VMEM((N, D), jnp.float32),
                        pltpu.VMEM((2, D), jnp.float32)],
    )(part, s, bc, ga, be, wk, wq, wv, ws, bk, bq, bv)


def _comb_fc_kernel(p_ref, s_ref, bc, ga, be, fw, fb, out_o, t_all, acc):
    j = pl.program_id(0)
    i = pl.program_id(1)

    @pl.when(j == 0)
    def _():
        _combine_stats(j, i, p_ref, s_ref, bc, t_all, acc)

    @pl.when(j == 1)
    def _():
        h = _bn_from(t_all, acc, i, ga, be)
        out_o[...] = jnp.dot(h, fw[...], preferred_element_type=jnp.float32) + fb[...]


@jax.jit
def _comb_fc(part, s, bc, ga, be, fw, fb):
    return pl.pallas_call(
        _comb_fc_kernel,
        grid=(2, NBLK),
        in_specs=[_pj_spec, _rowj_spec, _bj_spec, _bj_spec, _bj_spec,
                  _wj_spec, _bj_spec],
        out_specs=_rowo_spec,
        out_shape=jax.ShapeDtypeStruct((N, D), jnp.float32),
        scratch_shapes=[pltpu.VMEM((N, D), jnp.float32),
                        pltpu.VMEM((2, D), jnp.float32)],
    )(part, s, bc, ga, be, fw, fb)


def kernel(x, edge_index, emb, Wk, bk, Wq, bq, Wv, bv, Ws, bconv, gamma, beta,
           fcW, fcb):
    x = x.astype(jnp.int32)
    src = edge_index[0].astype(jnp.int32)
    dst = edge_index[1].astype(jnp.int32)
    r = lambda b: b.reshape(1, D)

    h0 = _emb_gather(emb, x)
    k1, q1, v1, s1 = _mm4(h0, Wk[0], Wq[0], Wv[0], Ws[0],
                          r(bk[0]), r(bq[0]), r(bv[0]))
    p1 = _edge_stage(k1, q1, v1, src, dst)
    k2, q2, v2, s2 = _comb_mm4(p1, s1, r(bconv[0]), r(gamma[0]), r(beta[0]),
                               Wk[1], Wq[1], Wv[1], Ws[1],
                               r(bk[1]), r(bq[1]), r(bv[1]))
    p2 = _edge_stage(k2, q2, v2, src, dst)
    return _comb_fc(p2, s2, r(bconv[1]), r(gamma[1]), r(beta[1]),
                    fcW, r(fcb))


# speedup vs baseline: 1.4649x; 1.2683x over previous
import functools

import jax
import jax.numpy as jnp
from jax import lax
from jax.experimental import pallas as pl
from jax.experimental.pallas import tpu as pltpu
from jax.experimental.pallas import tpu_sc as plsc

N = 10000
E = 320000
D = 128
NC = 2
NS = 16
NW = NC * NS

CH = 40
EPT = E // NW
NCH = EPT // CH
ZCH = 40
ZNCH = N // ZCH
ZPT = -(-ZNCH // NS)

ECH = 80
ENCH = N // ECH

_SC_MESH = plsc.VectorSubcoreMesh(core_axis_name="c", subcore_axis_name="s")


def _emb_body(emb_hbm, x_hbm, out_hbm, idx_v, rows_v, sem):
    c = lax.axis_index("c")
    s = lax.axis_index("s")
    wid = c * NS + s
    for i in range(4):
        cid = wid + i * NW

        @pl.when(cid < ENCH)
        def _():
            base = cid * ECH
            pltpu.sync_copy(x_hbm.at[pl.ds(base, ECH)], idx_v)
            pltpu.async_copy(emb_hbm.at[idx_v], rows_v, sem).wait()
            pltpu.sync_copy(rows_v, out_hbm.at[pl.ds(base, ECH)])


@jax.jit
def _emb_gather(emb, x):
    return pl.kernel(
        _emb_body,
        out_type=jax.ShapeDtypeStruct((N, D), jnp.float32),
        mesh=_SC_MESH,
        scratch_types=[
            pltpu.VMEM((ECH,), jnp.int32),
            pltpu.VMEM((ECH, D), jnp.float32),
            pltpu.SemaphoreType.DMA,
        ],
    )(emb, x)


def _edge_body(k_hbm, q_hbm, v_hbm, src_hbm, dst_hbm, part_hbm,
               sh_agg,
               ixs0, ixd0, ixs1, ixd1, ixs2, ixd2, ixs3, ixd3,
               kd0, qs0, vs0, kd1, qs1, vs1, m0, m1,
               rsem0, rsem1, isem0, isem1, isem2, isem3, ssem0, ssem1):
    c = lax.axis_index("c")
    s = lax.axis_index("s")
    wid = c * NS + s
    ebase = wid * EPT

    idx_s = (ixs0, ixs1, ixs2, ixs3)
    idx_d = (ixd0, ixd1, ixd2, ixd3)
    kd = (kd0, kd1)
    qs = (qs0, qs1)
    vs = (vs0, vs1)
    m = (m0, m1)
    rsems = (rsem0, rsem1)
    isems = (isem0, isem1, isem2, isem3)
    ssems = (ssem0, ssem1)

    zero16 = jnp.zeros((16,), jnp.float32)

    def zfill(r, carry):
        for j in range(D // 16):
            m0[r, pl.ds(j * 16, 16)] = zero16
        return carry

    lax.fori_loop(0, ZCH, zfill, 0)
    for z in range(ZPT):
        cid = s + z * NS

        @pl.when(cid < ZNCH)
        def _():
            pltpu.sync_copy(m0, sh_agg.at[pl.ds(cid * ZCH, ZCH)])

    plsc.subcore_barrier()

    def fetch_idx(ci, isl):
        base = ebase + ci * CH
        pltpu.async_copy(src_hbm.at[pl.ds(base, CH)], idx_s[isl], isems[isl])
        pltpu.async_copy(dst_hbm.at[pl.ds(base, CH)], idx_d[isl], isems[isl])

    def wait_idx(isl):
        pltpu.make_async_copy(src_hbm.at[pl.ds(0, CH)], idx_s[isl],
                              isems[isl]).wait()
        pltpu.make_async_copy(dst_hbm.at[pl.ds(0, CH)], idx_d[isl],
                              isems[isl]).wait()

    def fetch_rows(isl, rsl):
        pltpu.async_copy(k_hbm.at[idx_d[isl]], kd[rsl], rsems[rsl])
        pltpu.async_copy(q_hbm.at[idx_s[isl]], qs[rsl], rsems[rsl])
        pltpu.async_copy(v_hbm.at[idx_s[isl]], vs[rsl], rsems[rsl])

    def wait_scatter(rsl):
        pltpu.make_async_copy(m[rsl], sh_agg.at[idx_d[0]], ssems[rsl]).wait()

    def consume(isl, rsl):
        pltpu.make_async_copy(k_hbm.at[idx_d[isl]], kd[rsl], rsems[rsl]).wait()
        pltpu.make_async_copy(q_hbm.at[idx_s[isl]], qs[rsl], rsems[rsl]).wait()
        pltpu.make_async_copy(v_hbm.at[idx_s[isl]], vs[rsl], rsems[rsl]).wait()

        @plsc.parallel_loop(0, CH)
        def edge_body(e):
            for j in range(D // 16):
                sl = pl.ds(j * 16, 16)
                x = kd[rsl][e, sl] + qs[rsl][e, sl]
                eta = 1.0 / (1.0 + jnp.exp(-x))
                m[rsl][e, sl] = eta * vs[rsl][e, sl]

        pltpu.async_copy(m[rsl], sh_agg.at[idx_d[isl]], ssems[rsl], add=True)

    fetch_idx(0, 0)
    wait_idx(0)
    fetch_rows(0, 0)
    fetch_idx(1, 1)

    def outer4(t, carry):
        ci = t * 4
        wait_idx(1)
        fetch_rows(1, 1)

        @pl.when(t > 0)
        def _():
            wait_scatter(0)

        fetch_idx(ci + 2, 2)
        consume(0, 0)
        wait_idx(2)
        fetch_rows(2, 0)

        @pl.when(t > 0)
        def _():
            wait_scatter(1)

        fetch_idx(ci + 3, 3)
        consume(1, 1)
        wait_idx(3)
        fetch_rows(3, 1)
        wait_scatter(0)
        fetch_idx(ci + 4, 0)
        consume(2, 0)
        wait_idx(0)
        fetch_rows(0, 0)
        wait_scatter(1)
        fetch_idx(ci + 5, 1)
        consume(3, 1)
        return carry

    lax.fori_loop(0, (NCH - 2) // 4, outer4, 0)

    wait_idx(1)
    fetch_rows(1, 1)
    wait_scatter(0)
    consume(0, 0)
    wait_scatter(1)
    consume(1, 1)
    wait_scatter(0)
    wait_scatter(1)
    plsc.subcore_barrier()

    for z in range(ZPT):
        cid = s + z * NS

        @pl.when(cid < ZNCH)
        def _():
            row0 = cid * ZCH
            pltpu.sync_copy(sh_agg.at[pl.ds(row0, ZCH)], m0)
            pltpu.sync_copy(m0, part_hbm.at[c, pl.ds(row0, ZCH)])


@jax.jit
def _edge_stage(k, q, v, src, dst):
    return pl.kernel(
        _edge_body,
        out_type=jax.ShapeDtypeStruct((NC, N, D), jnp.float32),
        mesh=_SC_MESH,
        scratch_types=[
            pltpu.VMEM_SHARED((N, D), jnp.float32),
            pltpu.VMEM((CH,), jnp.int32),
            pltpu.VMEM((CH,), jnp.int32),
            pltpu.VMEM((CH,), jnp.int32),
            pltpu.VMEM((CH,), jnp.int32),
            pltpu.VMEM((CH,), jnp.int32),
            pltpu.VMEM((CH,), jnp.int32),
            pltpu.VMEM((CH,), jnp.int32),
            pltpu.VMEM((CH,), jnp.int32),
            pltpu.VMEM((CH, D), jnp.float32),
            pltpu.VMEM((CH, D), jnp.float32),
            pltpu.VMEM((CH, D), jnp.float32),
            pltpu.VMEM((CH, D), jnp.float32),
            pltpu.VMEM((CH, D), jnp.float32),
            pltpu.VMEM((CH, D), jnp.float32),
            pltpu.VMEM((CH, D), jnp.float32),
            pltpu.VMEM((CH, D), jnp.float32),
            pltpu.SemaphoreType.DMA,
            pltpu.SemaphoreType.DMA,
            pltpu.SemaphoreType.DMA,
            pltpu.SemaphoreType.DMA,
            pltpu.SemaphoreType.DMA,
            pltpu.SemaphoreType.DMA,
            pltpu.SemaphoreType.DMA,
            pltpu.SemaphoreType.DMA,
        ],
    )(k, q, v, src, dst)


BLK = 400
NBLK = N // BLK


def _mm4_kernel(h_ref, wk, wq, wv, ws, bk, bq, bv, k_o, q_o, v_o, s_o):
    h = h_ref[...]
    k_o[...] = jnp.dot(h, wk[...], preferred_element_type=jnp.float32) + bk[...]
    q_o[...] = jnp.dot(h, wq[...], preferred_element_type=jnp.float32) + bq[...]
    v_o[...] = jnp.dot(h, wv[...], preferred_element_type=jnp.float32) + bv[...]
    s_o[...] = jnp.dot(h, ws[...], preferred_element_type=jnp.float32)


def _mm4_bn_kernel(t_ref, mu, var, ga, be, wk, wq, wv, ws, bk, bq, bv,
                   k_o, q_o, v_o, s_o):
    t = t_ref[...]
    h = (t - mu[...]) * jax.lax.rsqrt(var[...] + 1e-5) * ga[...] + be[...]
    h = jnp.maximum(h, 0.0)
    k_o[...] = jnp.dot(h, wk[...], preferred_element_type=jnp.float32) + bk[...]
    q_o[...] = jnp.dot(h, wq[...], preferred_element_type=jnp.float32) + bq[...]
    v_o[...] = jnp.dot(h, wv[...], preferred_element_type=jnp.float32) + bv[...]
    s_o[...] = jnp.dot(h, ws[...], preferred_element_type=jnp.float32)


_row_spec = pl.BlockSpec((BLK, D), lambda i: (i, 0))
_w_spec = pl.BlockSpec((D, D), lambda i: (0, 0))
_b_spec = pl.BlockSpec((1, D), lambda i: (0, 0))
_out4 = [jax.ShapeDtypeStruct((N, D), jnp.float32)] * 4


@jax.jit
def _mm4(h, wk, wq, wv, ws, bk, bq, bv):
    return pl.pallas_call(
        _mm4_kernel,
        grid=(NBLK,),
        in_specs=[_row_spec, _w_spec, _w_spec, _w_spec, _w_spec,
                  _b_spec, _b_spec, _b_spec],
        out_specs=[_row_spec] * 4,
        out_shape=_out4,
    )(h, wk, wq, wv, ws, bk, bq, bv)


@jax.jit
def _mm4_bn(t, mu, var, ga, be, wk, wq, wv, ws, bk, bq, bv):
    return pl.pallas_call(
        _mm4_bn_kernel,
        grid=(NBLK,),
        in_specs=[_row_spec, _b_spec, _b_spec, _b_spec, _b_spec,
                  _w_spec, _w_spec, _w_spec, _w_spec,
                  _b_spec, _b_spec, _b_spec],
        out_specs=[_row_spec] * 4,
        out_shape=_out4,
    )(t, mu, var, ga, be, wk, wq, wv, ws, bk, bq, bv)


def _combine_stats(j, i, p_ref, s_ref, bc, t_all, acc):
    t = p_ref[0] + p_ref[1] + s_ref[...] + bc[...]
    t_all[pl.ds(i * BLK, BLK), :] = t
    psum = jnp.sum(t, axis=0, keepdims=True)
    psq = jnp.sum(t * t, axis=0, keepdims=True)

    @pl.when(i == 0)
    def _():
        acc[0:1] = psum
        acc[1:2] = psq

    @pl.when(i > 0)
    def _():
        acc[0:1] += psum
        acc[1:2] += psq


def _bn_from(t_all, acc, i, ga, be):
    mu = acc[0:1] / N
    var = acc[1:2] / N - mu * mu
    t = t_all[pl.ds(i * BLK, BLK), :]
    h = (t - mu) * jax.lax.rsqrt(var + 1e-5) * ga[...] + be[...]
    return jnp.maximum(h, 0.0)


def _comb_mm4_kernel(p_ref, s_ref, bc, ga, be, wk, wq, wv, ws, bk, bq, bv,
                     k_o, q_o, v_o, s_o, t_all, acc):
    j = pl.program_id(0)
    i = pl.program_id(1)

    @pl.when(j == 0)
    def _():
        _combine_stats(j, i, p_ref, s_ref, bc, t_all, acc)

    @pl.when(j == 1)
    def _():
        h = _bn_from(t_all, acc, i, ga, be)
        k_o[...] = jnp.dot(h, wk[...], preferred_element_type=jnp.float32) + bk[...]
        q_o[...] = jnp.dot(h, wq[...], preferred_element_type=jnp.float32) + bq[...]
        v_o[...] = jnp.dot(h, wv[...], preferred_element_type=jnp.float32) + bv[...]
        s_o[...] = jnp.dot(h, ws[...], preferred_element_type=jnp.float32)


_pj_spec = pl.BlockSpec((NC, BLK, D), lambda j, i: (0, i * (1 - j), 0))
_rowj_spec = pl.BlockSpec((BLK, D), lambda j, i: (i * (1 - j), 0))
_rowo_spec = pl.BlockSpec((BLK, D), lambda j, i: (i, 0))
_wj_spec = pl.BlockSpec((D, D), lambda j, i: (0, 0))
_bj_spec = pl.BlockSpec((1, D), lambda j, i: (0, 0))


@jax.jit
def _comb_mm4(part, s, bc, ga, be, wk, wq, wv, ws, bk, bq, bv):
    return pl.pallas_call(
        _comb_mm4_kernel,
        grid=(2, NBLK),
        in_specs=[_pj_spec, _rowj_spec, _bj_spec, _bj_spec, _bj_spec,
                  _wj_spec, _wj_spec, _wj_spec, _wj_spec,
                  _bj_spec, _bj_spec, _bj_spec],
        out_specs=[_rowo_spec] * 4,
        out_shape=_out4,
        scratch_shapes=[pltpu.VMEM((N, D), jnp.float32),
                        pltpu.VMEM((2, D), jnp.float32)],
    )(part, s, bc, ga, be, wk, wq, wv, ws, bk, bq, bv)


def _comb_fc_kernel(p_ref, s_ref, bc, ga, be, fw, fb, out_o, t_all, acc):
    j = pl.program_id(0)
    i = pl.program_id(1)

    @pl.when(j == 0)
    def _():
        _combine_stats(j, i, p_ref, s_ref, bc, t_all, acc)

    @pl.when(j == 1)
    def _():
        h = _bn_from(t_all, acc, i, ga, be)
        out_o[...] = jnp.dot(h, fw[...], preferred_element_type=jnp.float32) + fb[...]


@jax.jit
def _comb_fc(part, s, bc, ga, be, fw, fb):
    return pl.pallas_call(
        _comb_fc_kernel,
        grid=(2, NBLK),
        in_specs=[_pj_spec, _rowj_spec, _bj_spec, _bj_spec, _bj_spec,
                  _wj_spec, _bj_spec],
        out_specs=_rowo_spec,
        out_shape=jax.ShapeDtypeStruct((N, D), jnp.float32),
        scratch_shapes=[pltpu.VMEM((N, D), jnp.float32),
                        pltpu.VMEM((2, D), jnp.float32)],
    )(part, s, bc, ga, be, fw, fb)


def kernel(x, edge_index, emb, Wk, bk, Wq, bq, Wv, bv, Ws, bconv, gamma, beta,
           fcW, fcb):
    x = x.astype(jnp.int32)
    src = edge_index[0].astype(jnp.int32)
    dst = edge_index[1].astype(jnp.int32)
    r = lambda b: b.reshape(1, D)

    h0 = _emb_gather(emb, x)
    k1, q1, v1, s1 = _mm4(h0, Wk[0], Wq[0], Wv[0], Ws[0],
                          r(bk[0]), r(bq[0]), r(bv[0]))
    p1 = _edge_stage(k1, q1, v1, src, dst)
    k2, q2, v2, s2 = _comb_mm4(p1, s1, r(bconv[0]), r(gamma[0]), r(beta[0]),
                               Wk[1], Wq[1], Wv[1], Ws[1],
                               r(bk[1]), r(bq[1]), r(bv[1]))
    p2 = _edge_stage(k2, q2, v2, src, dst)
    return _comb_fc(p2, s2, r(bconv[1]), r(gamma[1]), r(beta[1]),
                    fcW, r(fcb))
